# Initial kernel scaffold; baseline (speedup 1.0000x reference)
import jax
import jax.numpy as jnp
from jax.experimental import pallas as pl


def kernel(rel_features, im_indices, Wq, bq, Wk, bk, Wv, bv, Wo, bo, ln1_g, ln1_b, W1, b1, W2, b2, ln2_g, ln2_b):
    raise NotImplementedError("write your pallas kernel here")


# speedup vs baseline: 6.1641x; 6.1641x over previous
import functools

import jax
import jax.numpy as jnp
import numpy as np
from jax.experimental import pallas as pl
from jax.experimental.pallas import tpu as pltpu

N_REL = 16384
N_FRAMES = 16
D_MODEL = 512
N_HEADS = 8
DH = D_MODEL // N_HEADS
D_FF = 2048
BLK = 512
NBLK = N_REL // BLK
SCALE = 1.0 / float(np.sqrt(DH))
NEG = -1e9


def _meta_kernel(seg_ref, kvs_ref, kve_ref):
    seg = seg_ref[...]
    f = jax.lax.broadcasted_iota(jnp.int32, (N_FRAMES, N_REL), 0)
    lt = (seg < f).astype(jnp.int32)
    le = (seg <= f).astype(jnp.int32)
    cl = jnp.sum(lt, axis=1, keepdims=True)
    ce = jnp.sum(le, axis=1, keepdims=True)
    blk = jax.lax.broadcasted_iota(jnp.int32, (N_FRAMES, 128), 1)
    inter = (cl < (blk + 1) * BLK) & (ce > blk * BLK)
    starts = jnp.where(inter, cl, jnp.int32(N_REL))
    ends = jnp.where(inter, ce, jnp.int32(0))
    start_tok = jnp.min(starts, axis=0, keepdims=True)
    end_tok = jnp.max(ends, axis=0, keepdims=True)
    kvs_ref[...] = jnp.clip(start_tok // BLK, 0, NBLK - 1)
    kve_ref[...] = jnp.clip((end_tok - 1) // BLK, 0, NBLK - 1)


def _compute_ranges(seg):
    out = pl.pallas_call(
        _meta_kernel,
        out_shape=(
            jax.ShapeDtypeStruct((1, 128), jnp.int32),
            jax.ShapeDtypeStruct((1, 128), jnp.int32),
        ),
    )(seg.reshape(1, N_REL).astype(jnp.int32))
    return out[0][0, :NBLK], out[1][0, :NBLK]


def _qkv_kernel(x_ref, w_ref, b_ref, q_ref, k_ref, v_ref):
    x = x_ref[...].astype(jnp.bfloat16)
    y = jax.lax.dot_general(x, w_ref[...], (((1,), (0,)), ((), ())),
                            preferred_element_type=jnp.float32)
    y = y + b_ref[...]
    q_ref[...] = y[:, 0:D_MODEL]
    k_ref[...] = y[:, D_MODEL:2 * D_MODEL]
    v_ref[...] = y[:, 2 * D_MODEL:3 * D_MODEL]


def _qkv_proj(x, w_qkv, b_qkv):
    return pl.pallas_call(
        _qkv_kernel,
        grid=(NBLK,),
        in_specs=[
            pl.BlockSpec((BLK, D_MODEL), lambda i: (i, 0)),
            pl.BlockSpec((D_MODEL, 3 * D_MODEL), lambda i: (0, 0)),
            pl.BlockSpec((1, 3 * D_MODEL), lambda i: (0, 0)),
        ],
        out_specs=(
            pl.BlockSpec((BLK, D_MODEL), lambda i: (i, 0)),
            pl.BlockSpec((BLK, D_MODEL), lambda i: (i, 0)),
            pl.BlockSpec((BLK, D_MODEL), lambda i: (i, 0)),
        ),
        out_shape=(
            jax.ShapeDtypeStruct((N_REL, D_MODEL), jnp.float32),
            jax.ShapeDtypeStruct((N_REL, D_MODEL), jnp.float32),
            jax.ShapeDtypeStruct((N_REL, D_MODEL), jnp.float32),
        ),
    )(x, w_qkv, b_qkv)


def _attn_kernel(kvs_ref, kve_ref, q_ref, k_ref, v_ref, sq_ref, sk_ref,
                 o_ref, acc_ref, m_ref, l_ref):
    i = pl.program_id(0)
    j = pl.program_id(1)

    @pl.when(j == 0)
    def _init():
        acc_ref[...] = jnp.zeros_like(acc_ref)
        m_ref[...] = jnp.full_like(m_ref, -1e30)
        l_ref[...] = jnp.zeros_like(l_ref)

    @pl.when((j >= kvs_ref[i]) & (j <= kve_ref[i]))
    def _body():
        sq = sq_ref[0, 0, :]
        sk = sk_ref[0, 0, :]
        mask = sq[:, None] == sk[None, :]
        for h in range(N_HEADS):
            qh = q_ref[:, h * DH:(h + 1) * DH].astype(jnp.bfloat16)
            kh = k_ref[:, h * DH:(h + 1) * DH].astype(jnp.bfloat16)
            vh = v_ref[:, h * DH:(h + 1) * DH].astype(jnp.bfloat16)
            s = jax.lax.dot_general(qh, kh, (((1,), (1,)), ((), ())),
                                    preferred_element_type=jnp.float32)
            s = jnp.where(mask, s * SCALE, NEG)
            m_prev = m_ref[h][:, 0:1]
            l_prev = l_ref[h][:, 0:1]
            m_new = jnp.maximum(m_prev, jnp.max(s, axis=1, keepdims=True))
            p = jnp.where(mask, jnp.exp(s - m_new), 0.0)
            alpha = jnp.exp(m_prev - m_new)
            l_new = l_prev * alpha + jnp.sum(p, axis=1, keepdims=True)
            pv = jax.lax.dot_general(p.astype(jnp.bfloat16), vh,
                                     (((1,), (0,)), ((), ())),
                                     preferred_element_type=jnp.float32)
            acc_ref[h] = acc_ref[h] * alpha + pv
            m_ref[h] = jnp.broadcast_to(m_new, (BLK, 128))
            l_ref[h] = jnp.broadcast_to(l_new, (BLK, 128))

    @pl.when(j == NBLK - 1)
    def _flush():
        for h in range(N_HEADS):
            inv_l = 1.0 / l_ref[h][:, 0:1]
            o_ref[:, h * DH:(h + 1) * DH] = acc_ref[h] * inv_l


def _attention(q, k, v, seg3, kvs, kve):
    def qmap(i, j, kvs_ref, kve_ref):
        return (i, 0)

    def kvmap(i, j, kvs_ref, kve_ref):
        return (jnp.clip(j, kvs_ref[i], kve_ref[i]), 0)

    def sqmap(i, j, kvs_ref, kve_ref):
        return (i, 0, 0)

    def skmap(i, j, kvs_ref, kve_ref):
        return (jnp.clip(j, kvs_ref[i], kve_ref[i]), 0, 0)

    grid_spec = pltpu.PrefetchScalarGridSpec(
        num_scalar_prefetch=2,
        grid=(NBLK, NBLK),
        in_specs=[
            pl.BlockSpec((BLK, D_MODEL), qmap),
            pl.BlockSpec((BLK, D_MODEL), kvmap),
            pl.BlockSpec((BLK, D_MODEL), kvmap),
            pl.BlockSpec((1, 1, BLK), sqmap),
            pl.BlockSpec((1, 1, BLK), skmap),
        ],
        out_specs=pl.BlockSpec((BLK, D_MODEL), qmap),
        scratch_shapes=[
            pltpu.VMEM((N_HEADS, BLK, DH), jnp.float32),
            pltpu.VMEM((N_HEADS, BLK, 128), jnp.float32),
            pltpu.VMEM((N_HEADS, BLK, 128), jnp.float32),
        ],
    )
    return pl.pallas_call(
        _attn_kernel,
        grid_spec=grid_spec,
        out_shape=jax.ShapeDtypeStruct((N_REL, D_MODEL), jnp.float32),
        compiler_params=pltpu.CompilerParams(
            dimension_semantics=("arbitrary", "arbitrary"),
        ),
    )(kvs, kve, q, k, v, seg3, seg3)


def _ln(x, g, b):
    m = jnp.mean(x, axis=-1, keepdims=True)
    c = x - m
    v = jnp.mean(c * c, axis=-1, keepdims=True)
    return c * jax.lax.rsqrt(v + 1e-5) * g + b


def _tail_kernel(ctx_ref, x_ref, wo_ref, bo_ref, g1_ref, b1n_ref,
                 w1_ref, b1_ref, w2_ref, b2_ref, g2_ref, b2n_ref, o_ref):
    ctx = ctx_ref[...].astype(jnp.bfloat16)
    t = jax.lax.dot_general(ctx, wo_ref[...], (((1,), (0,)), ((), ())),
                            preferred_element_type=jnp.float32)
    t = x_ref[...] + t + bo_ref[...]
    x1 = _ln(t, g1_ref[...], b1n_ref[...])
    h = jax.lax.dot_general(x1.astype(jnp.bfloat16), w1_ref[...],
                            (((1,), (0,)), ((), ())),
                            preferred_element_type=jnp.float32)
    h = jnp.maximum(h + b1_ref[...], 0.0)
    ff = jax.lax.dot_general(h.astype(jnp.bfloat16), w2_ref[...],
                             (((1,), (0,)), ((), ())),
                             preferred_element_type=jnp.float32)
    ff = ff + b2_ref[...]
    o_ref[...] = _ln(x1 + ff, g2_ref[...], b2n_ref[...])


def _tail(ctx, x, wo, bo, g1, b1n, w1, b1, w2, b2, g2, b2n):
    row = lambda i: (i, 0)
    full = lambda i: (0, 0)
    return pl.pallas_call(
        _tail_kernel,
        grid=(NBLK,),
        in_specs=[
            pl.BlockSpec((BLK, D_MODEL), row),
            pl.BlockSpec((BLK, D_MODEL), row),
            pl.BlockSpec((D_MODEL, D_MODEL), full),
            pl.BlockSpec((1, D_MODEL), full),
            pl.BlockSpec((1, D_MODEL), full),
            pl.BlockSpec((1, D_MODEL), full),
            pl.BlockSpec((D_MODEL, D_FF), full),
            pl.BlockSpec((1, D_FF), full),
            pl.BlockSpec((D_FF, D_MODEL), full),
            pl.BlockSpec((1, D_MODEL), full),
            pl.BlockSpec((1, D_MODEL), full),
            pl.BlockSpec((1, D_MODEL), full),
        ],
        out_specs=pl.BlockSpec((BLK, D_MODEL), row),
        out_shape=jax.ShapeDtypeStruct((N_REL, D_MODEL), jnp.float32),
    )(ctx, x, wo, bo, g1, b1n, w1, b1, w2, b2, g2, b2n)


def kernel(rel_features, im_indices, Wq, bq, Wk, bk, Wv, bv, Wo, bo,
           ln1_g, ln1_b, W1, b1, W2, b2, ln2_g, ln2_b):
    seg = im_indices.astype(jnp.int32)
    kvs, kve = _compute_ranges(seg)
    seg3 = seg.reshape(NBLK, 1, BLK)

    w_qkv = jnp.concatenate([Wq, Wk, Wv], axis=1).astype(jnp.bfloat16)
    b_qkv = jnp.concatenate([bq, bk, bv]).reshape(1, 3 * D_MODEL)
    q, k, v = _qkv_proj(rel_features, w_qkv, b_qkv)

    ctx = _attention(q, k, v, seg3, kvs, kve)

    out = _tail(ctx, rel_features,
                Wo.astype(jnp.bfloat16), bo.reshape(1, D_MODEL),
                ln1_g.reshape(1, D_MODEL), ln1_b.reshape(1, D_MODEL),
                W1.astype(jnp.bfloat16), b1.reshape(1, D_FF),
                W2.astype(jnp.bfloat16), b2.reshape(1, D_MODEL),
                ln2_g.reshape(1, D_MODEL), ln2_b.reshape(1, D_MODEL))
    return out


# speedup vs baseline: 8.7141x; 1.4137x over previous
import functools

import jax
import jax.numpy as jnp
import numpy as np
from jax.experimental import pallas as pl
from jax.experimental.pallas import tpu as pltpu

N_REL = 16384
N_FRAMES = 16
D_MODEL = 512
N_HEADS = 8
DH = D_MODEL // N_HEADS
D_FF = 2048
BLK = 512
NBLK = N_REL // BLK
SCALE = 1.0 / float(np.sqrt(DH))
NEG = -1e9


def _meta_kernel(seg_ref, kvs_ref, kve_ref):
    seg = seg_ref[...]
    f = jax.lax.broadcasted_iota(jnp.int32, (N_FRAMES, N_REL), 0)
    lt = (seg < f).astype(jnp.int32)
    le = (seg <= f).astype(jnp.int32)
    cl = jnp.sum(lt, axis=1, keepdims=True)
    ce = jnp.sum(le, axis=1, keepdims=True)
    blk = jax.lax.broadcasted_iota(jnp.int32, (N_FRAMES, 128), 1)
    inter = (cl < (blk + 1) * BLK) & (ce > blk * BLK)
    starts = jnp.where(inter, cl, jnp.int32(N_REL))
    ends = jnp.where(inter, ce, jnp.int32(0))
    start_tok = jnp.min(starts, axis=0, keepdims=True)
    end_tok = jnp.max(ends, axis=0, keepdims=True)
    kvs_ref[...] = jnp.clip(start_tok // BLK, 0, NBLK - 1)
    kve_ref[...] = jnp.clip((end_tok - 1) // BLK, 0, NBLK - 1)


def _compute_ranges(seg):
    out = pl.pallas_call(
        _meta_kernel,
        out_shape=(
            jax.ShapeDtypeStruct((1, 128), jnp.int32),
            jax.ShapeDtypeStruct((1, 128), jnp.int32),
        ),
    )(seg.reshape(1, N_REL).astype(jnp.int32))
    return out[0][0, :NBLK], out[1][0, :NBLK]


def _qkv_kernel(x_ref, w_ref, b_ref, q_ref, k_ref, v_ref):
    x = x_ref[...].astype(jnp.bfloat16)
    y = jax.lax.dot_general(x, w_ref[...], (((1,), (0,)), ((), ())),
                            preferred_element_type=jnp.float32)
    y = y + b_ref[...]
    ones = jnp.ones((BLK, 1), jnp.bfloat16)
    zeros = jnp.zeros((BLK, 128 - DH - 1), jnp.bfloat16)
    for h in range(N_HEADS):
        q_ref[h] = (y[:, h * DH:(h + 1) * DH] * SCALE).astype(jnp.bfloat16)
        k_ref[h] = y[:, D_MODEL + h * DH:D_MODEL + (h + 1) * DH].astype(
            jnp.bfloat16)
        vh = y[:, 2 * D_MODEL + h * DH:2 * D_MODEL + (h + 1) * DH].astype(
            jnp.bfloat16)
        v_ref[h] = jnp.concatenate([vh, ones, zeros], axis=1)


def _qkv_proj(x, w_qkv, b_qkv):
    return pl.pallas_call(
        _qkv_kernel,
        grid=(NBLK,),
        in_specs=[
            pl.BlockSpec((BLK, D_MODEL), lambda i: (i, 0)),
            pl.BlockSpec((D_MODEL, 3 * D_MODEL), lambda i: (0, 0)),
            pl.BlockSpec((1, 3 * D_MODEL), lambda i: (0, 0)),
        ],
        out_specs=(
            pl.BlockSpec((N_HEADS, BLK, DH), lambda i: (0, i, 0)),
            pl.BlockSpec((N_HEADS, BLK, DH), lambda i: (0, i, 0)),
            pl.BlockSpec((N_HEADS, BLK, 128), lambda i: (0, i, 0)),
        ),
        out_shape=(
            jax.ShapeDtypeStruct((N_HEADS, N_REL, DH), jnp.bfloat16),
            jax.ShapeDtypeStruct((N_HEADS, N_REL, DH), jnp.bfloat16),
            jax.ShapeDtypeStruct((N_HEADS, N_REL, 128), jnp.bfloat16),
        ),
    )(x, w_qkv, b_qkv)


def _attn_kernel(kvs_ref, kve_ref, q_ref, k_ref, v_ref, sq_ref, sk_ref,
                 o_ref, acc_ref, m_ref):
    i = pl.program_id(0)
    j = pl.program_id(1)

    @pl.when(j == 0)
    def _init():
        acc_ref[...] = jnp.zeros_like(acc_ref)
        m_ref[...] = jnp.full_like(m_ref, -1e30)

    @pl.when((j >= kvs_ref[i]) & (j <= kve_ref[i]))
    def _body():
        sq = sq_ref[0, 0, :]
        sk = sk_ref[0, 0, :]
        mask = sq[:, None] == sk[None, :]
        bias = jnp.where(mask, 0.0, NEG)
        for h in range(N_HEADS):
            s = jax.lax.dot_general(q_ref[h], k_ref[h],
                                    (((1,), (1,)), ((), ())),
                                    preferred_element_type=jnp.float32)
            s = s + bias
            m_prev = m_ref[h][:, 0:1]
            m_new = jnp.maximum(m_prev, jnp.max(s, axis=1, keepdims=True))
            p = jnp.exp((s - m_new).astype(jnp.bfloat16))
            pv = jax.lax.dot_general(p, v_ref[h], (((1,), (0,)), ((), ())),
                                     preferred_element_type=jnp.float32)
            alpha = jnp.exp(m_prev - m_new)
            acc_ref[h] = acc_ref[h] * alpha + pv
            m_ref[h] = jnp.broadcast_to(m_new, (BLK, 128))

    @pl.when(j == NBLK - 1)
    def _flush():
        for h in range(N_HEADS):
            a = acc_ref[h]
            inv_l = 1.0 / a[:, DH:DH + 1]
            o_ref[:, h * DH:(h + 1) * DH] = a[:, 0:DH] * inv_l


def _attention(q, k, v, seg3, kvs, kve):
    def qmap(i, j, kvs_ref, kve_ref):
        return (0, i, 0)

    def kvmap(i, j, kvs_ref, kve_ref):
        return (0, jnp.clip(j, kvs_ref[i], kve_ref[i]), 0)

    def omap(i, j, kvs_ref, kve_ref):
        return (i, 0)

    def sqmap(i, j, kvs_ref, kve_ref):
        return (i, 0, 0)

    def skmap(i, j, kvs_ref, kve_ref):
        return (jnp.clip(j, kvs_ref[i], kve_ref[i]), 0, 0)

    grid_spec = pltpu.PrefetchScalarGridSpec(
        num_scalar_prefetch=2,
        grid=(NBLK, NBLK),
        in_specs=[
            pl.BlockSpec((N_HEADS, BLK, DH), qmap),
            pl.BlockSpec((N_HEADS, BLK, DH), kvmap),
            pl.BlockSpec((N_HEADS, BLK, 128), kvmap),
            pl.BlockSpec((1, 1, BLK), sqmap),
            pl.BlockSpec((1, 1, BLK), skmap),
        ],
        out_specs=pl.BlockSpec((BLK, D_MODEL), omap),
        scratch_shapes=[
            pltpu.VMEM((N_HEADS, BLK, 128), jnp.float32),
            pltpu.VMEM((N_HEADS, BLK, 128), jnp.float32),
        ],
    )
    return pl.pallas_call(
        _attn_kernel,
        grid_spec=grid_spec,
        out_shape=jax.ShapeDtypeStruct((N_REL, D_MODEL), jnp.float32),
        compiler_params=pltpu.CompilerParams(
            dimension_semantics=("arbitrary", "arbitrary"),
        ),
    )(kvs, kve, q, k, v, seg3, seg3)


def _ln(x, g, b):
    m = jnp.mean(x, axis=-1, keepdims=True)
    c = x - m
    v = jnp.mean(c * c, axis=-1, keepdims=True)
    return c * jax.lax.rsqrt(v + 1e-5) * g + b


def _tail_kernel(ctx_ref, x_ref, wo_ref, bo_ref, g1_ref, b1n_ref,
                 w1_ref, b1_ref, w2_ref, b2_ref, g2_ref, b2n_ref, o_ref):
    ctx = ctx_ref[...].astype(jnp.bfloat16)
    t = jax.lax.dot_general(ctx, wo_ref[...], (((1,), (0,)), ((), ())),
                            preferred_element_type=jnp.float32)
    t = x_ref[...] + t + bo_ref[...]
    x1 = _ln(t, g1_ref[...], b1n_ref[...])
    h = jax.lax.dot_general(x1.astype(jnp.bfloat16), w1_ref[...],
                            (((1,), (0,)), ((), ())),
                            preferred_element_type=jnp.float32)
    h = jnp.maximum(h + b1_ref[...], 0.0)
    ff = jax.lax.dot_general(h.astype(jnp.bfloat16), w2_ref[...],
                             (((1,), (0,)), ((), ())),
                             preferred_element_type=jnp.float32)
    ff = ff + b2_ref[...]
    o_ref[...] = _ln(x1 + ff, g2_ref[...], b2n_ref[...])


def _tail(ctx, x, wo, bo, g1, b1n, w1, b1, w2, b2, g2, b2n):
    row = lambda i: (i, 0)
    full = lambda i: (0, 0)
    return pl.pallas_call(
        _tail_kernel,
        grid=(NBLK,),
        in_specs=[
            pl.BlockSpec((BLK, D_MODEL), row),
            pl.BlockSpec((BLK, D_MODEL), row),
            pl.BlockSpec((D_MODEL, D_MODEL), full),
            pl.BlockSpec((1, D_MODEL), full),
            pl.BlockSpec((1, D_MODEL), full),
            pl.BlockSpec((1, D_MODEL), full),
            pl.BlockSpec((D_MODEL, D_FF), full),
            pl.BlockSpec((1, D_FF), full),
            pl.BlockSpec((D_FF, D_MODEL), full),
            pl.BlockSpec((1, D_MODEL), full),
            pl.BlockSpec((1, D_MODEL), full),
            pl.BlockSpec((1, D_MODEL), full),
        ],
        out_specs=pl.BlockSpec((BLK, D_MODEL), row),
        out_shape=jax.ShapeDtypeStruct((N_REL, D_MODEL), jnp.float32),
    )(ctx, x, wo, bo, g1, b1n, w1, b1, w2, b2, g2, b2n)


def kernel(rel_features, im_indices, Wq, bq, Wk, bk, Wv, bv, Wo, bo,
           ln1_g, ln1_b, W1, b1, W2, b2, ln2_g, ln2_b):
    seg = im_indices.astype(jnp.int32)
    kvs, kve = _compute_ranges(seg)
    seg3 = seg.reshape(NBLK, 1, BLK)

    w_qkv = jnp.concatenate([Wq, Wk, Wv], axis=1).astype(jnp.bfloat16)
    b_qkv = jnp.concatenate([bq, bk, bv]).reshape(1, 3 * D_MODEL)
    q, k, v = _qkv_proj(rel_features, w_qkv, b_qkv)

    ctx = _attention(q, k, v, seg3, kvs, kve)

    out = _tail(ctx, rel_features,
                Wo.astype(jnp.bfloat16), bo.reshape(1, D_MODEL),
                ln1_g.reshape(1, D_MODEL), ln1_b.reshape(1, D_MODEL),
                W1.astype(jnp.bfloat16), b1.reshape(1, D_FF),
                W2.astype(jnp.bfloat16), b2.reshape(1, D_MODEL),
                ln2_g.reshape(1, D_MODEL), ln2_b.reshape(1, D_MODEL))
    return out


# speedup vs baseline: 9.5923x; 1.1008x over previous
import functools

import jax
import jax.numpy as jnp
import numpy as np
from jax.experimental import pallas as pl
from jax.experimental.pallas import tpu as pltpu

N_REL = 16384
N_FRAMES = 16
D_MODEL = 512
N_HEADS = 8
DH = D_MODEL // N_HEADS
D_FF = 2048
BLK = 512
NBLK = N_REL // BLK
SCALE = 1.0 / float(np.sqrt(DH))
NEG = -1e9


def _meta_kernel(seg_ref, kvs_ref, kve_ref):
    seg = seg_ref[...]
    f = jax.lax.broadcasted_iota(jnp.int32, (N_FRAMES, N_REL), 0)
    lt = (seg < f).astype(jnp.int32)
    le = (seg <= f).astype(jnp.int32)
    cl = jnp.sum(lt, axis=1, keepdims=True)
    ce = jnp.sum(le, axis=1, keepdims=True)
    blk = jax.lax.broadcasted_iota(jnp.int32, (N_FRAMES, 128), 1)
    inter = (cl < (blk + 1) * BLK) & (ce > blk * BLK)
    starts = jnp.where(inter, cl, jnp.int32(N_REL))
    ends = jnp.where(inter, ce, jnp.int32(0))
    start_tok = jnp.min(starts, axis=0, keepdims=True)
    end_tok = jnp.max(ends, axis=0, keepdims=True)
    kvs_ref[...] = jnp.clip(start_tok // BLK, 0, NBLK - 1)
    kve_ref[...] = jnp.clip((end_tok - 1) // BLK, 0, NBLK - 1)


def _compute_ranges(seg):
    out = pl.pallas_call(
        _meta_kernel,
        out_shape=(
            jax.ShapeDtypeStruct((1, 128), jnp.int32),
            jax.ShapeDtypeStruct((1, 128), jnp.int32),
        ),
    )(seg.reshape(1, N_REL).astype(jnp.int32))
    return out[0][0, :NBLK], out[1][0, :NBLK]


def _qkv_kernel(x_ref, w_ref, b_ref, q_ref, k_ref, v_ref):
    x = x_ref[...].astype(jnp.bfloat16)
    y = jax.lax.dot_general(x, w_ref[...], (((1,), (0,)), ((), ())),
                            preferred_element_type=jnp.float32)
    y = y + b_ref[...]
    ones = jnp.ones((BLK, 1), jnp.bfloat16)
    zeros = jnp.zeros((BLK, 128 - DH - 1), jnp.bfloat16)
    for h in range(N_HEADS):
        q_ref[h] = (y[:, h * DH:(h + 1) * DH] * SCALE).astype(jnp.bfloat16)
        k_ref[h] = y[:, D_MODEL + h * DH:D_MODEL + (h + 1) * DH].astype(
            jnp.bfloat16)
        vh = y[:, 2 * D_MODEL + h * DH:2 * D_MODEL + (h + 1) * DH].astype(
            jnp.bfloat16)
        v_ref[h] = jnp.concatenate([vh, ones, zeros], axis=1)


def _qkv_proj(x, w_qkv, b_qkv):
    return pl.pallas_call(
        _qkv_kernel,
        grid=(NBLK,),
        in_specs=[
            pl.BlockSpec((BLK, D_MODEL), lambda i: (i, 0)),
            pl.BlockSpec((D_MODEL, 3 * D_MODEL), lambda i: (0, 0)),
            pl.BlockSpec((1, 3 * D_MODEL), lambda i: (0, 0)),
        ],
        out_specs=(
            pl.BlockSpec((N_HEADS, BLK, DH), lambda i: (0, i, 0)),
            pl.BlockSpec((N_HEADS, BLK, DH), lambda i: (0, i, 0)),
            pl.BlockSpec((N_HEADS, BLK, 128), lambda i: (0, i, 0)),
        ),
        out_shape=(
            jax.ShapeDtypeStruct((N_HEADS, N_REL, DH), jnp.bfloat16),
            jax.ShapeDtypeStruct((N_HEADS, N_REL, DH), jnp.bfloat16),
            jax.ShapeDtypeStruct((N_HEADS, N_REL, 128), jnp.bfloat16),
        ),
    )(x, w_qkv, b_qkv)


def _attn_kernel(kvs_ref, kve_ref, q_ref, k_ref, v_ref, sq_ref, sk_ref,
                 o_ref, acc_ref, m_ref):
    i = pl.program_id(0)
    j = pl.program_id(1)

    @pl.when(j == 0)
    def _init():
        acc_ref[...] = jnp.zeros_like(acc_ref)
        m_ref[...] = jnp.full_like(m_ref, -1e30)

    @pl.when((j >= kvs_ref[i]) & (j <= kve_ref[i]))
    def _body():
        sq = sq_ref[0, 0, :]
        sk = sk_ref[0, 0, :]
        mask = sq[:, None] == sk[None, :]
        bias = jnp.where(mask, 0.0, NEG).astype(jnp.bfloat16)
        for h in range(N_HEADS):
            s = jax.lax.dot_general(q_ref[h], k_ref[h],
                                    (((1,), (1,)), ((), ())),
                                    preferred_element_type=jnp.float32)
            s = s.astype(jnp.bfloat16) + bias
            m_prev = m_ref[h][:, 0:1]
            m_new = jnp.maximum(m_prev, jnp.max(s, axis=1, keepdims=True))
            p = jnp.exp(s - m_new)
            pv = jax.lax.dot_general(p, v_ref[h], (((1,), (0,)), ((), ())),
                                     preferred_element_type=jnp.float32)
            alpha = jnp.exp((m_prev - m_new).astype(jnp.float32))
            acc_ref[h] = acc_ref[h] * alpha + pv
            m_ref[h] = jnp.broadcast_to(m_new, (BLK, 128))

    @pl.when(j == NBLK - 1)
    def _flush():
        for h in range(N_HEADS):
            a = acc_ref[h]
            inv_l = 1.0 / a[:, DH:DH + 1]
            o_ref[:, h * DH:(h + 1) * DH] = a[:, 0:DH] * inv_l


def _attention(q, k, v, seg3, kvs, kve):
    def qmap(i, j, kvs_ref, kve_ref):
        return (0, i, 0)

    def kvmap(i, j, kvs_ref, kve_ref):
        return (0, jnp.clip(j, kvs_ref[i], kve_ref[i]), 0)

    def omap(i, j, kvs_ref, kve_ref):
        return (i, 0)

    def sqmap(i, j, kvs_ref, kve_ref):
        return (i, 0, 0)

    def skmap(i, j, kvs_ref, kve_ref):
        return (jnp.clip(j, kvs_ref[i], kve_ref[i]), 0, 0)

    grid_spec = pltpu.PrefetchScalarGridSpec(
        num_scalar_prefetch=2,
        grid=(NBLK, NBLK),
        in_specs=[
            pl.BlockSpec((N_HEADS, BLK, DH), qmap),
            pl.BlockSpec((N_HEADS, BLK, DH), kvmap),
            pl.BlockSpec((N_HEADS, BLK, 128), kvmap),
            pl.BlockSpec((1, 1, BLK), sqmap),
            pl.BlockSpec((1, 1, BLK), skmap),
        ],
        out_specs=pl.BlockSpec((BLK, D_MODEL), omap),
        scratch_shapes=[
            pltpu.VMEM((N_HEADS, BLK, 128), jnp.float32),
            pltpu.VMEM((N_HEADS, BLK, 128), jnp.bfloat16),
        ],
    )
    return pl.pallas_call(
        _attn_kernel,
        grid_spec=grid_spec,
        out_shape=jax.ShapeDtypeStruct((N_REL, D_MODEL), jnp.float32),
        compiler_params=pltpu.CompilerParams(
            dimension_semantics=("arbitrary", "arbitrary"),
        ),
    )(kvs, kve, q, k, v, seg3, seg3)


def _ln(x, g, b):
    m = jnp.mean(x, axis=-1, keepdims=True)
    c = x - m
    v = jnp.mean(c * c, axis=-1, keepdims=True)
    return c * jax.lax.rsqrt(v + 1e-5) * g + b


def _tail_kernel(ctx_ref, x_ref, wo_ref, bo_ref, g1_ref, b1n_ref,
                 w1_ref, b1_ref, w2_ref, b2_ref, g2_ref, b2n_ref, o_ref):
    ctx = ctx_ref[...].astype(jnp.bfloat16)
    t = jax.lax.dot_general(ctx, wo_ref[...], (((1,), (0,)), ((), ())),
                            preferred_element_type=jnp.float32)
    t = x_ref[...] + t + bo_ref[...]
    x1 = _ln(t, g1_ref[...], b1n_ref[...])
    h = jax.lax.dot_general(x1.astype(jnp.bfloat16), w1_ref[...],
                            (((1,), (0,)), ((), ())),
                            preferred_element_type=jnp.float32)
    h = jnp.maximum(h + b1_ref[...], 0.0)
    ff = jax.lax.dot_general(h.astype(jnp.bfloat16), w2_ref[...],
                             (((1,), (0,)), ((), ())),
                             preferred_element_type=jnp.float32)
    ff = ff + b2_ref[...]
    o_ref[...] = _ln(x1 + ff, g2_ref[...], b2n_ref[...])


def _tail(ctx, x, wo, bo, g1, b1n, w1, b1, w2, b2, g2, b2n):
    row = lambda i: (i, 0)
    full = lambda i: (0, 0)
    return pl.pallas_call(
        _tail_kernel,
        grid=(NBLK,),
        in_specs=[
            pl.BlockSpec((BLK, D_MODEL), row),
            pl.BlockSpec((BLK, D_MODEL), row),
            pl.BlockSpec((D_MODEL, D_MODEL), full),
            pl.BlockSpec((1, D_MODEL), full),
            pl.BlockSpec((1, D_MODEL), full),
            pl.BlockSpec((1, D_MODEL), full),
            pl.BlockSpec((D_MODEL, D_FF), full),
            pl.BlockSpec((1, D_FF), full),
            pl.BlockSpec((D_FF, D_MODEL), full),
            pl.BlockSpec((1, D_MODEL), full),
            pl.BlockSpec((1, D_MODEL), full),
            pl.BlockSpec((1, D_MODEL), full),
        ],
        out_specs=pl.BlockSpec((BLK, D_MODEL), row),
        out_shape=jax.ShapeDtypeStruct((N_REL, D_MODEL), jnp.float32),
    )(ctx, x, wo, bo, g1, b1n, w1, b1, w2, b2, g2, b2n)


def kernel(rel_features, im_indices, Wq, bq, Wk, bk, Wv, bv, Wo, bo,
           ln1_g, ln1_b, W1, b1, W2, b2, ln2_g, ln2_b):
    seg = im_indices.astype(jnp.int32)
    kvs, kve = _compute_ranges(seg)
    seg3 = seg.reshape(NBLK, 1, BLK)

    w_qkv = jnp.concatenate([Wq, Wk, Wv], axis=1).astype(jnp.bfloat16)
    b_qkv = jnp.concatenate([bq, bk, bv]).reshape(1, 3 * D_MODEL)
    q, k, v = _qkv_proj(rel_features, w_qkv, b_qkv)

    ctx = _attention(q, k, v, seg3, kvs, kve)

    out = _tail(ctx, rel_features,
                Wo.astype(jnp.bfloat16), bo.reshape(1, D_MODEL),
                ln1_g.reshape(1, D_MODEL), ln1_b.reshape(1, D_MODEL),
                W1.astype(jnp.bfloat16), b1.reshape(1, D_FF),
                W2.astype(jnp.bfloat16), b2.reshape(1, D_MODEL),
                ln2_g.reshape(1, D_MODEL), ln2_b.reshape(1, D_MODEL))
    return out


# speedup vs baseline: 11.0761x; 1.1547x over previous
import functools

import jax
import jax.numpy as jnp
import numpy as np
from jax.experimental import pallas as pl
from jax.experimental.pallas import tpu as pltpu

N_REL = 16384
N_FRAMES = 16
D_MODEL = 512
N_HEADS = 8
DH = D_MODEL // N_HEADS
D_FF = 2048
BLK = 512
NBLK = N_REL // BLK
SCALE = 1.0 / float(np.sqrt(DH))
NEG = -1e9


NPMAX = NBLK * NBLK


def _meta_kernel(seg_ref, pi_ref, pj_ref, np_ref):
    seg = seg_ref[...]
    f = jax.lax.broadcasted_iota(jnp.int32, (N_FRAMES, N_REL), 0)
    lt = (seg < f).astype(jnp.int32)
    le = (seg <= f).astype(jnp.int32)
    cl = jnp.sum(lt, axis=1, keepdims=True)
    ce = jnp.sum(le, axis=1, keepdims=True)
    clr = jax.lax.transpose(jnp.broadcast_to(cl, (N_FRAMES, 128)), (1, 0))
    cer = jax.lax.transpose(jnp.broadcast_to(ce, (N_FRAMES, 128)), (1, 0))
    bi = jax.lax.broadcasted_iota(jnp.int32, (128, N_FRAMES), 0)
    inter = (clr < (bi + 1) * BLK) & (cer > bi * BLK)
    starts = jnp.where(inter, clr, jnp.int32(N_REL))
    ends = jnp.where(inter, cer, jnp.int32(0))
    kvs128 = jnp.clip(jnp.min(starts, axis=1, keepdims=True) // BLK,
                      0, NBLK - 1)
    kve128 = jnp.clip((jnp.max(ends, axis=1, keepdims=True) - 1) // BLK,
                      0, NBLK - 1)
    rowi = jax.lax.broadcasted_iota(jnp.int32, (128, 1), 0)
    span128 = jnp.where(rowi < NBLK, kve128 - kvs128 + 1, 0)
    kvs = kvs128[0:NBLK, :]
    span = span128[0:NBLK, :]
    spanT = jax.lax.transpose(jnp.broadcast_to(span128, (128, 128)),
                              (1, 0))[0:NBLK, :]
    li = jax.lax.broadcasted_iota(jnp.int32, (NBLK, 128), 1)
    si = jax.lax.broadcasted_iota(jnp.int32, (NBLK, 128), 0)
    off = jnp.sum(jnp.where(li < si, spanT, 0),
                  axis=1, keepdims=True)
    npair = jnp.sum(spanT[0:1, :], axis=1, keepdims=True)
    pp = jax.lax.broadcasted_iota(jnp.int32, (NBLK, NPMAX), 1)
    off_b = jnp.broadcast_to(off, (NBLK, NPMAX))
    offn_b = jnp.broadcast_to(off + span, (NBLK, NPMAX))
    pair_i = jnp.sum((offn_b <= pp).astype(jnp.int32), axis=0,
                     keepdims=True)
    onehot = ((off_b <= pp) & (pp < offn_b)).astype(jnp.int32)
    kvs_sel = jnp.sum(onehot * jnp.broadcast_to(kvs, (NBLK, NPMAX)),
                      axis=0, keepdims=True)
    off_sel = jnp.sum(onehot * off_b, axis=0, keepdims=True)
    pair_j = kvs_sel + pp[0:1, :] - off_sel
    pi_ref[...] = jnp.clip(pair_i, 0, NBLK - 1)
    pj_ref[...] = jnp.clip(pair_j, 0, NBLK - 1)
    np_ref[...] = jnp.broadcast_to(npair, (1, 128))


def _compute_pairs(seg):
    out = pl.pallas_call(
        _meta_kernel,
        out_shape=(
            jax.ShapeDtypeStruct((1, NPMAX), jnp.int32),
            jax.ShapeDtypeStruct((1, NPMAX), jnp.int32),
            jax.ShapeDtypeStruct((1, 128), jnp.int32),
        ),
    )(seg.reshape(1, N_REL).astype(jnp.int32))
    return out[0][0], out[1][0], out[2][0, 0]


def _qkv_kernel(x_ref, w_ref, b_ref, q_ref, k_ref, v_ref):
    x = x_ref[...].astype(jnp.bfloat16)
    y = jax.lax.dot_general(x, w_ref[...], (((1,), (0,)), ((), ())),
                            preferred_element_type=jnp.float32)
    y = y + b_ref[...]
    ones = jnp.ones((BLK, 1), jnp.bfloat16)
    zeros = jnp.zeros((BLK, 128 - DH - 1), jnp.bfloat16)
    for h in range(N_HEADS):
        q_ref[h] = (y[:, h * DH:(h + 1) * DH] * SCALE).astype(jnp.bfloat16)
        k_ref[h] = y[:, D_MODEL + h * DH:D_MODEL + (h + 1) * DH].astype(
            jnp.bfloat16)
        vh = y[:, 2 * D_MODEL + h * DH:2 * D_MODEL + (h + 1) * DH].astype(
            jnp.bfloat16)
        v_ref[h] = jnp.concatenate([vh, ones, zeros], axis=1)


def _qkv_proj(x, w_qkv, b_qkv):
    return pl.pallas_call(
        _qkv_kernel,
        grid=(NBLK,),
        in_specs=[
            pl.BlockSpec((BLK, D_MODEL), lambda i: (i, 0)),
            pl.BlockSpec((D_MODEL, 3 * D_MODEL), lambda i: (0, 0)),
            pl.BlockSpec((1, 3 * D_MODEL), lambda i: (0, 0)),
        ],
        out_specs=(
            pl.BlockSpec((N_HEADS, BLK, DH), lambda i: (0, i, 0)),
            pl.BlockSpec((N_HEADS, BLK, DH), lambda i: (0, i, 0)),
            pl.BlockSpec((N_HEADS, BLK, 128), lambda i: (0, i, 0)),
        ),
        out_shape=(
            jax.ShapeDtypeStruct((N_HEADS, N_REL, DH), jnp.bfloat16),
            jax.ShapeDtypeStruct((N_HEADS, N_REL, DH), jnp.bfloat16),
            jax.ShapeDtypeStruct((N_HEADS, N_REL, 128), jnp.bfloat16),
        ),
    )(x, w_qkv, b_qkv)


def _attn_kernel(pi_ref, pj_ref, q_ref, k_ref, v_ref, sq_ref, sk_ref,
                 o_ref, acc_ref, m_ref):
    p = pl.program_id(0)
    i = pi_ref[p]

    @pl.when((p == 0) | (pi_ref[jnp.maximum(p - 1, 0)] != i))
    def _init():
        acc_ref[...] = jnp.zeros_like(acc_ref)
        m_ref[...] = jnp.full_like(m_ref, -1e30)

    sq = sq_ref[0, 0, :]
    sk = sk_ref[0, 0, :]
    mask = sq[:, None] == sk[None, :]
    bias = jnp.where(mask, 0.0, NEG).astype(jnp.bfloat16)
    for h in range(N_HEADS):
        s = jax.lax.dot_general(q_ref[h], k_ref[h],
                                (((1,), (1,)), ((), ())),
                                preferred_element_type=jnp.float32)
        s = s.astype(jnp.bfloat16) + bias
        m_prev = m_ref[h][:, 0:1]
        m_new = jnp.maximum(m_prev, jnp.max(s, axis=1, keepdims=True))
        p_ = jnp.exp(s - m_new)
        pv = jax.lax.dot_general(p_, v_ref[h], (((1,), (0,)), ((), ())),
                                 preferred_element_type=jnp.float32)
        alpha = jnp.exp((m_prev - m_new).astype(jnp.float32))
        acc_ref[h] = acc_ref[h] * alpha + pv
        m_ref[h] = jnp.broadcast_to(m_new, (BLK, 128))

    @pl.when((p == pl.num_programs(0) - 1)
             | (pi_ref[jnp.minimum(p + 1, NPMAX - 1)] != i))
    def _flush():
        for h in range(N_HEADS):
            a = acc_ref[h]
            inv_l = 1.0 / a[:, DH:DH + 1]
            o_ref[:, h * DH:(h + 1) * DH] = a[:, 0:DH] * inv_l


def _attention(q, k, v, seg3, pair_i, pair_j, n_pairs):
    def qmap(p, pi_ref, pj_ref):
        return (0, pi_ref[p], 0)

    def kvmap(p, pi_ref, pj_ref):
        return (0, pj_ref[p], 0)

    def omap(p, pi_ref, pj_ref):
        return (pi_ref[p], 0)

    def sqmap(p, pi_ref, pj_ref):
        return (pi_ref[p], 0, 0)

    def skmap(p, pi_ref, pj_ref):
        return (pj_ref[p], 0, 0)

    grid_spec = pltpu.PrefetchScalarGridSpec(
        num_scalar_prefetch=2,
        grid=(n_pairs,),
        in_specs=[
            pl.BlockSpec((N_HEADS, BLK, DH), qmap),
            pl.BlockSpec((N_HEADS, BLK, DH), kvmap),
            pl.BlockSpec((N_HEADS, BLK, 128), kvmap),
            pl.BlockSpec((1, 1, BLK), sqmap),
            pl.BlockSpec((1, 1, BLK), skmap),
        ],
        out_specs=pl.BlockSpec((BLK, D_MODEL), omap),
        scratch_shapes=[
            pltpu.VMEM((N_HEADS, BLK, 128), jnp.float32),
            pltpu.VMEM((N_HEADS, BLK, 128), jnp.bfloat16),
        ],
    )
    return pl.pallas_call(
        _attn_kernel,
        grid_spec=grid_spec,
        out_shape=jax.ShapeDtypeStruct((N_REL, D_MODEL), jnp.float32),
        compiler_params=pltpu.CompilerParams(
            dimension_semantics=("arbitrary",),
        ),
    )(pair_i, pair_j, q, k, v, seg3, seg3)


def _ln(x, g, b):
    m = jnp.mean(x, axis=-1, keepdims=True)
    c = x - m
    v = jnp.mean(c * c, axis=-1, keepdims=True)
    return c * jax.lax.rsqrt(v + 1e-5) * g + b


def _tail_kernel(ctx_ref, x_ref, wo_ref, bo_ref, g1_ref, b1n_ref,
                 w1_ref, b1_ref, w2_ref, b2_ref, g2_ref, b2n_ref, o_ref):
    ctx = ctx_ref[...].astype(jnp.bfloat16)
    t = jax.lax.dot_general(ctx, wo_ref[...], (((1,), (0,)), ((), ())),
                            preferred_element_type=jnp.float32)
    t = x_ref[...] + t + bo_ref[...]
    x1 = _ln(t, g1_ref[...], b1n_ref[...])
    h = jax.lax.dot_general(x1.astype(jnp.bfloat16), w1_ref[...],
                            (((1,), (0,)), ((), ())),
                            preferred_element_type=jnp.float32)
    h = jnp.maximum(h + b1_ref[...], 0.0)
    ff = jax.lax.dot_general(h.astype(jnp.bfloat16), w2_ref[...],
                             (((1,), (0,)), ((), ())),
                             preferred_element_type=jnp.float32)
    ff = ff + b2_ref[...]
    o_ref[...] = _ln(x1 + ff, g2_ref[...], b2n_ref[...])


def _tail(ctx, x, wo, bo, g1, b1n, w1, b1, w2, b2, g2, b2n):
    row = lambda i: (i, 0)
    full = lambda i: (0, 0)
    return pl.pallas_call(
        _tail_kernel,
        grid=(NBLK,),
        in_specs=[
            pl.BlockSpec((BLK, D_MODEL), row),
            pl.BlockSpec((BLK, D_MODEL), row),
            pl.BlockSpec((D_MODEL, D_MODEL), full),
            pl.BlockSpec((1, D_MODEL), full),
            pl.BlockSpec((1, D_MODEL), full),
            pl.BlockSpec((1, D_MODEL), full),
            pl.BlockSpec((D_MODEL, D_FF), full),
            pl.BlockSpec((1, D_FF), full),
            pl.BlockSpec((D_FF, D_MODEL), full),
            pl.BlockSpec((1, D_MODEL), full),
            pl.BlockSpec((1, D_MODEL), full),
            pl.BlockSpec((1, D_MODEL), full),
        ],
        out_specs=pl.BlockSpec((BLK, D_MODEL), row),
        out_shape=jax.ShapeDtypeStruct((N_REL, D_MODEL), jnp.float32),
    )(ctx, x, wo, bo, g1, b1n, w1, b1, w2, b2, g2, b2n)


def kernel(rel_features, im_indices, Wq, bq, Wk, bk, Wv, bv, Wo, bo,
           ln1_g, ln1_b, W1, b1, W2, b2, ln2_g, ln2_b):
    seg = im_indices.astype(jnp.int32)
    pair_i, pair_j, n_pairs = _compute_pairs(seg)
    seg3 = seg.reshape(NBLK, 1, BLK)

    w_qkv = jnp.concatenate([Wq, Wk, Wv], axis=1).astype(jnp.bfloat16)
    b_qkv = jnp.concatenate([bq, bk, bv]).reshape(1, 3 * D_MODEL)
    q, k, v = _qkv_proj(rel_features, w_qkv, b_qkv)

    ctx = _attention(q, k, v, seg3, pair_i, pair_j, n_pairs)

    out = _tail(ctx, rel_features,
                Wo.astype(jnp.bfloat16), bo.reshape(1, D_MODEL),
                ln1_g.reshape(1, D_MODEL), ln1_b.reshape(1, D_MODEL),
                W1.astype(jnp.bfloat16), b1.reshape(1, D_FF),
                W2.astype(jnp.bfloat16), b2.reshape(1, D_MODEL),
                ln2_g.reshape(1, D_MODEL), ln2_b.reshape(1, D_MODEL))
    return out


# speedup vs baseline: 12.2311x; 1.1043x over previous
import functools

import jax
import jax.numpy as jnp
import numpy as np
from jax.experimental import pallas as pl
from jax.experimental.pallas import tpu as pltpu

N_REL = 16384
N_FRAMES = 16
D_MODEL = 512
N_HEADS = 8
DH = D_MODEL // N_HEADS
D_FF = 2048
BLK = 512
NBLK = N_REL // BLK
SCALE = 1.0 / float(np.sqrt(DH))
NEG = -1e9


NPMAX = NBLK * NBLK


def _meta_kernel(seg_ref, pi_ref, pj_ref, np_ref):
    seg = seg_ref[...]
    f = jax.lax.broadcasted_iota(jnp.int32, (N_FRAMES, N_REL), 0)
    lt = (seg < f).astype(jnp.int32)
    le = (seg <= f).astype(jnp.int32)
    cl = jnp.sum(lt, axis=1, keepdims=True)
    ce = jnp.sum(le, axis=1, keepdims=True)
    clr = jax.lax.transpose(jnp.broadcast_to(cl, (N_FRAMES, 128)), (1, 0))
    cer = jax.lax.transpose(jnp.broadcast_to(ce, (N_FRAMES, 128)), (1, 0))
    bi = jax.lax.broadcasted_iota(jnp.int32, (128, N_FRAMES), 0)
    inter = (clr < (bi + 1) * BLK) & (cer > bi * BLK)
    starts = jnp.where(inter, clr, jnp.int32(N_REL))
    ends = jnp.where(inter, cer, jnp.int32(0))
    kvs128 = jnp.clip(jnp.min(starts, axis=1, keepdims=True) // BLK,
                      0, NBLK - 1)
    kve128 = jnp.clip((jnp.max(ends, axis=1, keepdims=True) - 1) // BLK,
                      0, NBLK - 1)
    rowi = jax.lax.broadcasted_iota(jnp.int32, (128, 1), 0)
    span128 = jnp.where(rowi < NBLK, kve128 - kvs128 + 1, 0)
    kvs = kvs128[0:NBLK, :]
    span = span128[0:NBLK, :]
    spanT = jax.lax.transpose(jnp.broadcast_to(span128, (128, 128)),
                              (1, 0))[0:NBLK, :]
    li = jax.lax.broadcasted_iota(jnp.int32, (NBLK, 128), 1)
    si = jax.lax.broadcasted_iota(jnp.int32, (NBLK, 128), 0)
    off = jnp.sum(jnp.where(li < si, spanT, 0),
                  axis=1, keepdims=True)
    npair = jnp.sum(spanT[0:1, :], axis=1, keepdims=True)
    pp = jax.lax.broadcasted_iota(jnp.int32, (NBLK, NPMAX), 1)
    off_b = jnp.broadcast_to(off, (NBLK, NPMAX))
    offn_b = jnp.broadcast_to(off + span, (NBLK, NPMAX))
    pair_i = jnp.sum((offn_b <= pp).astype(jnp.int32), axis=0,
                     keepdims=True)
    onehot = ((off_b <= pp) & (pp < offn_b)).astype(jnp.int32)
    kvs_sel = jnp.sum(onehot * jnp.broadcast_to(kvs, (NBLK, NPMAX)),
                      axis=0, keepdims=True)
    off_sel = jnp.sum(onehot * off_b, axis=0, keepdims=True)
    pair_j = kvs_sel + pp[0:1, :] - off_sel
    pi_ref[...] = jnp.clip(pair_i, 0, NBLK - 1)
    pj_ref[...] = jnp.clip(pair_j, 0, NBLK - 1)
    np_ref[...] = jnp.broadcast_to(npair, (1, 128))


def _compute_pairs(seg):
    out = pl.pallas_call(
        _meta_kernel,
        out_shape=(
            jax.ShapeDtypeStruct((1, NPMAX), jnp.int32),
            jax.ShapeDtypeStruct((1, NPMAX), jnp.int32),
            jax.ShapeDtypeStruct((1, 128), jnp.int32),
        ),
    )(seg.reshape(1, N_REL).astype(jnp.int32))
    return out[0][0], out[1][0], out[2][0, 0]


def _qkv_kernel(x_ref, w_ref, b_ref, q_ref, k_ref, v_ref):
    x = x_ref[...].astype(jnp.bfloat16)
    y = jax.lax.dot_general(x, w_ref[...], (((1,), (0,)), ((), ())),
                            preferred_element_type=jnp.float32)
    y = y + b_ref[...]
    ones = jnp.ones((BLK, 1), jnp.bfloat16)
    zeros = jnp.zeros((BLK, 128 - DH - 1), jnp.bfloat16)
    for h in range(N_HEADS):
        q_ref[h] = (y[:, h * DH:(h + 1) * DH] * SCALE).astype(jnp.bfloat16)
        k_ref[h] = y[:, D_MODEL + h * DH:D_MODEL + (h + 1) * DH].astype(
            jnp.bfloat16)
        vh = y[:, 2 * D_MODEL + h * DH:2 * D_MODEL + (h + 1) * DH].astype(
            jnp.bfloat16)
        v_ref[h] = jnp.concatenate([vh, ones, zeros], axis=1)


def _qkv_proj(x, w_qkv, b_qkv):
    return pl.pallas_call(
        _qkv_kernel,
        grid=(NBLK,),
        in_specs=[
            pl.BlockSpec((BLK, D_MODEL), lambda i: (i, 0)),
            pl.BlockSpec((D_MODEL, 3 * D_MODEL), lambda i: (0, 0)),
            pl.BlockSpec((1, 3 * D_MODEL), lambda i: (0, 0)),
        ],
        out_specs=(
            pl.BlockSpec((N_HEADS, BLK, DH), lambda i: (0, i, 0)),
            pl.BlockSpec((N_HEADS, BLK, DH), lambda i: (0, i, 0)),
            pl.BlockSpec((N_HEADS, BLK, 128), lambda i: (0, i, 0)),
        ),
        out_shape=(
            jax.ShapeDtypeStruct((N_HEADS, N_REL, DH), jnp.bfloat16),
            jax.ShapeDtypeStruct((N_HEADS, N_REL, DH), jnp.bfloat16),
            jax.ShapeDtypeStruct((N_HEADS, N_REL, 128), jnp.bfloat16),
        ),
    )(x, w_qkv, b_qkv)


def _attn_kernel(pi_ref, pj_ref, q_ref, k_ref, v_ref, sq_ref, sk_ref,
                 x_ref, wo_ref, bo_ref, g1_ref, b1n_ref, w1_ref, b1_ref,
                 w2_ref, b2_ref, g2_ref, b2n_ref,
                 o_ref, acc_ref, m_ref):
    p = pl.program_id(0)
    i = pi_ref[p]
    first = (p == 0) | (pi_ref[jnp.maximum(p - 1, 0)] != i)

    sq = sq_ref[0, 0, :]
    sk = sk_ref[0, 0, :]
    mask = sq[:, None] == sk[None, :]
    bias = jnp.where(mask, 0.0, NEG).astype(jnp.bfloat16)

    @pl.when(first)
    def _first_pair():
        for h in range(N_HEADS):
            s = jax.lax.dot_general(q_ref[h], k_ref[h],
                                    (((1,), (1,)), ((), ())),
                                    preferred_element_type=jnp.float32)
            s = s.astype(jnp.bfloat16) + bias
            m_new = jnp.max(s, axis=1, keepdims=True)
            p_ = jnp.exp(s - m_new)
            pv = jax.lax.dot_general(p_, v_ref[h], (((1,), (0,)), ((), ())),
                                     preferred_element_type=jnp.float32)
            acc_ref[h] = pv
            m_ref[h] = jnp.broadcast_to(m_new, (BLK, 128))

    @pl.when(jnp.logical_not(first))
    def _next_pair():
        for h in range(N_HEADS):
            s = jax.lax.dot_general(q_ref[h], k_ref[h],
                                    (((1,), (1,)), ((), ())),
                                    preferred_element_type=jnp.float32)
            s = s.astype(jnp.bfloat16) + bias
            m_prev = m_ref[h][:, 0:1]
            m_new = jnp.maximum(m_prev, jnp.max(s, axis=1, keepdims=True))
            p_ = jnp.exp(s - m_new)
            pv = jax.lax.dot_general(p_, v_ref[h], (((1,), (0,)), ((), ())),
                                     preferred_element_type=jnp.float32)
            alpha = jnp.exp((m_prev - m_new).astype(jnp.float32))
            acc_ref[h] = acc_ref[h] * alpha + pv
            m_ref[h] = jnp.broadcast_to(m_new, (BLK, 128))

    @pl.when((p == pl.num_programs(0) - 1)
             | (pi_ref[jnp.minimum(p + 1, NPMAX - 1)] != i))
    def _flush():
        parts = []
        for h in range(N_HEADS):
            a = acc_ref[h]
            inv_l = 1.0 / a[:, DH:DH + 1]
            parts.append((a[:, 0:DH] * inv_l).astype(jnp.bfloat16))
        ctx = jnp.concatenate(parts, axis=1)
        t = jax.lax.dot_general(ctx, wo_ref[...], (((1,), (0,)), ((), ())),
                                preferred_element_type=jnp.float32)
        t = x_ref[...] + t + bo_ref[...]
        x1 = _ln(t, g1_ref[...], b1n_ref[...])
        hh = jax.lax.dot_general(x1.astype(jnp.bfloat16), w1_ref[...],
                                 (((1,), (0,)), ((), ())),
                                 preferred_element_type=jnp.float32)
        hh = jnp.maximum(hh + b1_ref[...], 0.0)
        ff = jax.lax.dot_general(hh.astype(jnp.bfloat16), w2_ref[...],
                                 (((1,), (0,)), ((), ())),
                                 preferred_element_type=jnp.float32)
        ff = ff + b2_ref[...]
        o_ref[...] = _ln(x1 + ff, g2_ref[...], b2n_ref[...])


def _attention(q, k, v, seg3, pair_i, pair_j, n_pairs, x, wo, bo,
               g1, b1n, w1, b1, w2, b2, g2, b2n):
    def qmap(p, pi_ref, pj_ref):
        return (0, pi_ref[p], 0)

    def kvmap(p, pi_ref, pj_ref):
        return (0, pj_ref[p], 0)

    def omap(p, pi_ref, pj_ref):
        return (pi_ref[p], 0)

    def sqmap(p, pi_ref, pj_ref):
        return (pi_ref[p], 0, 0)

    def skmap(p, pi_ref, pj_ref):
        return (pj_ref[p], 0, 0)

    def full(p, pi_ref, pj_ref):
        return (0, 0)

    grid_spec = pltpu.PrefetchScalarGridSpec(
        num_scalar_prefetch=2,
        grid=(n_pairs,),
        in_specs=[
            pl.BlockSpec((N_HEADS, BLK, DH), qmap),
            pl.BlockSpec((N_HEADS, BLK, DH), kvmap),
            pl.BlockSpec((N_HEADS, BLK, 128), kvmap),
            pl.BlockSpec((1, 1, BLK), sqmap),
            pl.BlockSpec((1, 1, BLK), skmap),
            pl.BlockSpec((BLK, D_MODEL), omap),
            pl.BlockSpec((D_MODEL, D_MODEL), full),
            pl.BlockSpec((1, D_MODEL), full),
            pl.BlockSpec((1, D_MODEL), full),
            pl.BlockSpec((1, D_MODEL), full),
            pl.BlockSpec((D_MODEL, D_FF), full),
            pl.BlockSpec((1, D_FF), full),
            pl.BlockSpec((D_FF, D_MODEL), full),
            pl.BlockSpec((1, D_MODEL), full),
            pl.BlockSpec((1, D_MODEL), full),
            pl.BlockSpec((1, D_MODEL), full),
        ],
        out_specs=pl.BlockSpec((BLK, D_MODEL), omap),
        scratch_shapes=[
            pltpu.VMEM((N_HEADS, BLK, 128), jnp.float32),
            pltpu.VMEM((N_HEADS, BLK, 128), jnp.bfloat16),
        ],
    )
    return pl.pallas_call(
        _attn_kernel,
        grid_spec=grid_spec,
        out_shape=jax.ShapeDtypeStruct((N_REL, D_MODEL), jnp.float32),
        compiler_params=pltpu.CompilerParams(
            dimension_semantics=("arbitrary",),
        ),
    )(pair_i, pair_j, q, k, v, seg3, seg3, x, wo, bo,
      g1, b1n, w1, b1, w2, b2, g2, b2n)


def _ln(x, g, b):
    m = jnp.mean(x, axis=-1, keepdims=True)
    c = x - m
    v = jnp.mean(c * c, axis=-1, keepdims=True)
    return c * jax.lax.rsqrt(v + 1e-5) * g + b


def kernel(rel_features, im_indices, Wq, bq, Wk, bk, Wv, bv, Wo, bo,
           ln1_g, ln1_b, W1, b1, W2, b2, ln2_g, ln2_b):
    seg = im_indices.astype(jnp.int32)
    pair_i, pair_j, n_pairs = _compute_pairs(seg)
    seg3 = seg.reshape(NBLK, 1, BLK)

    w_qkv = jnp.concatenate([Wq, Wk, Wv], axis=1).astype(jnp.bfloat16)
    b_qkv = jnp.concatenate([bq, bk, bv]).reshape(1, 3 * D_MODEL)
    q, k, v = _qkv_proj(rel_features, w_qkv, b_qkv)

    out = _attention(q, k, v, seg3, pair_i, pair_j, n_pairs,
                     rel_features,
                     Wo.astype(jnp.bfloat16), bo.reshape(1, D_MODEL),
                     ln1_g.reshape(1, D_MODEL), ln1_b.reshape(1, D_MODEL),
                     W1.astype(jnp.bfloat16), b1.reshape(1, D_FF),
                     W2.astype(jnp.bfloat16), b2.reshape(1, D_MODEL),
                     ln2_g.reshape(1, D_MODEL), ln2_b.reshape(1, D_MODEL))
    return out


# speedup vs baseline: 12.5578x; 1.0267x over previous
import functools

import jax
import jax.numpy as jnp
import numpy as np
from jax.experimental import pallas as pl
from jax.experimental.pallas import tpu as pltpu

N_REL = 16384
N_FRAMES = 16
D_MODEL = 512
N_HEADS = 8
DH = D_MODEL // N_HEADS
D_FF = 2048
BLK = 512
NBLK = N_REL // BLK
SCALE = 1.0 / float(np.sqrt(DH))
NEG = -1e9


NPMAX = NBLK * NBLK


def _meta_kernel(seg_ref, pi_ref, pj_ref, pf_ref, np_ref):
    seg = seg_ref[...]
    f = jax.lax.broadcasted_iota(jnp.int32, (N_FRAMES, N_REL), 0)
    lt = (seg < f).astype(jnp.int32)
    le = (seg <= f).astype(jnp.int32)
    cl = jnp.sum(lt, axis=1, keepdims=True)
    ce = jnp.sum(le, axis=1, keepdims=True)
    clr = jax.lax.transpose(jnp.broadcast_to(cl, (N_FRAMES, 128)), (1, 0))
    cer = jax.lax.transpose(jnp.broadcast_to(ce, (N_FRAMES, 128)), (1, 0))
    bi = jax.lax.broadcasted_iota(jnp.int32, (128, N_FRAMES), 0)
    inter = (clr < (bi + 1) * BLK) & (cer > bi * BLK)
    starts = jnp.where(inter, clr, jnp.int32(N_REL))
    ends = jnp.where(inter, cer, jnp.int32(0))
    kvs128 = jnp.clip(jnp.min(starts, axis=1, keepdims=True) // BLK,
                      0, NBLK - 1)
    kve128 = jnp.clip((jnp.max(ends, axis=1, keepdims=True) - 1) // BLK,
                      0, NBLK - 1)
    rowi = jax.lax.broadcasted_iota(jnp.int32, (128, 1), 0)
    span128 = jnp.where(rowi < NBLK, kve128 - kvs128 + 1, 0)
    kvs = kvs128[0:NBLK, :]
    span = span128[0:NBLK, :]
    spanT = jax.lax.transpose(jnp.broadcast_to(span128, (128, 128)),
                              (1, 0))[0:NBLK, :]
    li = jax.lax.broadcasted_iota(jnp.int32, (NBLK, 128), 1)
    si = jax.lax.broadcasted_iota(jnp.int32, (NBLK, 128), 0)
    off = jnp.sum(jnp.where(li < si, spanT, 0),
                  axis=1, keepdims=True)
    npair = jnp.sum(spanT[0:1, :], axis=1, keepdims=True)
    pp = jax.lax.broadcasted_iota(jnp.int32, (NBLK, NPMAX), 1)
    off_b = jnp.broadcast_to(off, (NBLK, NPMAX))
    offn_b = jnp.broadcast_to(off + span, (NBLK, NPMAX))
    pair_i = jnp.sum((offn_b <= pp).astype(jnp.int32), axis=0,
                     keepdims=True)
    onehot = ((off_b <= pp) & (pp < offn_b)).astype(jnp.int32)
    kvs_sel = jnp.sum(onehot * jnp.broadcast_to(kvs, (NBLK, NPMAX)),
                      axis=0, keepdims=True)
    off_sel = jnp.sum(onehot * off_b, axis=0, keepdims=True)
    pair_j = kvs_sel + pp[0:1, :] - off_sel
    pair_i = jnp.clip(pair_i, 0, NBLK - 1)
    pair_j = jnp.clip(pair_j, 0, NBLK - 1)
    cont = (clr <= bi * BLK) & (cer >= (bi + 1) * BLK)
    fi = jax.lax.broadcasted_iota(jnp.int32, (128, N_FRAMES), 1)
    pure128 = jnp.max(cont.astype(jnp.int32), axis=1, keepdims=True)
    fr128 = jnp.sum(jnp.where(cont, fi, 0), axis=1, keepdims=True)
    pure_b = jnp.broadcast_to(pure128[0:NBLK, :], (NBLK, NPMAX))
    fr_b = jnp.broadcast_to(fr128[0:NBLK, :], (NBLK, NPMAX))
    sub_i = jax.lax.broadcasted_iota(jnp.int32, (NBLK, NPMAX), 0)
    oj = (jnp.broadcast_to(pair_j, (NBLK, NPMAX)) == sub_i).astype(jnp.int32)
    pure_i = jnp.sum(onehot * pure_b, axis=0, keepdims=True)
    fr_i = jnp.sum(onehot * fr_b, axis=0, keepdims=True)
    pure_j = jnp.sum(oj * pure_b, axis=0, keepdims=True)
    fr_j = jnp.sum(oj * fr_b, axis=0, keepdims=True)
    pair_full = ((pure_i == 1) & (pure_j == 1)
                 & (fr_i == fr_j)).astype(jnp.int32)
    pi_ref[...] = pair_i
    pj_ref[...] = pair_j
    pf_ref[...] = pair_full
    np_ref[...] = jnp.broadcast_to(npair, (1, 128))


def _compute_pairs(seg):
    out = pl.pallas_call(
        _meta_kernel,
        out_shape=(
            jax.ShapeDtypeStruct((1, NPMAX), jnp.int32),
            jax.ShapeDtypeStruct((1, NPMAX), jnp.int32),
            jax.ShapeDtypeStruct((1, NPMAX), jnp.int32),
            jax.ShapeDtypeStruct((1, 128), jnp.int32),
        ),
    )(seg.reshape(1, N_REL).astype(jnp.int32))
    return out[0][0], out[1][0], out[2][0], out[3][0, 0]


def _qkv_kernel(x_ref, w_ref, b_ref, q_ref, k_ref, v_ref):
    x = x_ref[...].astype(jnp.bfloat16)
    y = jax.lax.dot_general(x, w_ref[...], (((1,), (0,)), ((), ())),
                            preferred_element_type=jnp.float32)
    y = y + b_ref[...]
    ones = jnp.ones((BLK, 1), jnp.bfloat16)
    zeros = jnp.zeros((BLK, 128 - DH - 1), jnp.bfloat16)
    for h in range(N_HEADS):
        q_ref[h] = (y[:, h * DH:(h + 1) * DH] * SCALE).astype(jnp.bfloat16)
        k_ref[h] = y[:, D_MODEL + h * DH:D_MODEL + (h + 1) * DH].astype(
            jnp.bfloat16)
        vh = y[:, 2 * D_MODEL + h * DH:2 * D_MODEL + (h + 1) * DH].astype(
            jnp.bfloat16)
        v_ref[h] = jnp.concatenate([vh, ones, zeros], axis=1)


def _qkv_proj(x, w_qkv, b_qkv):
    return pl.pallas_call(
        _qkv_kernel,
        grid=(NBLK,),
        in_specs=[
            pl.BlockSpec((BLK, D_MODEL), lambda i: (i, 0)),
            pl.BlockSpec((D_MODEL, 3 * D_MODEL), lambda i: (0, 0)),
            pl.BlockSpec((1, 3 * D_MODEL), lambda i: (0, 0)),
        ],
        out_specs=(
            pl.BlockSpec((N_HEADS, BLK, DH), lambda i: (0, i, 0)),
            pl.BlockSpec((N_HEADS, BLK, DH), lambda i: (0, i, 0)),
            pl.BlockSpec((N_HEADS, BLK, 128), lambda i: (0, i, 0)),
        ),
        out_shape=(
            jax.ShapeDtypeStruct((N_HEADS, N_REL, DH), jnp.bfloat16),
            jax.ShapeDtypeStruct((N_HEADS, N_REL, DH), jnp.bfloat16),
            jax.ShapeDtypeStruct((N_HEADS, N_REL, 128), jnp.bfloat16),
        ),
    )(x, w_qkv, b_qkv)


def _attn_kernel(pi_ref, pj_ref, pf_ref, q_ref, k_ref, v_ref, sq_ref, sk_ref,
                 x_ref, wo_ref, bo_ref, g1_ref, b1n_ref, w1_ref, b1_ref,
                 w2_ref, b2_ref, g2_ref, b2n_ref,
                 o_ref, acc_ref, m_ref):
    p = pl.program_id(0)
    i = pi_ref[p]
    first = (p == 0) | (pi_ref[jnp.maximum(p - 1, 0)] != i)
    full = pf_ref[p] != 0

    def _make_bias():
        sq = sq_ref[0, 0, :]
        sk = sk_ref[0, 0, :]
        mask = sq[:, None] == sk[None, :]
        return jnp.where(mask, 0.0, NEG).astype(jnp.bfloat16)

    def _run(is_first, bias):
        for h in range(N_HEADS):
            s = jax.lax.dot_general(q_ref[h], k_ref[h],
                                    (((1,), (1,)), ((), ())),
                                    preferred_element_type=jnp.float32)
            s = s.astype(jnp.bfloat16)
            if bias is not None:
                s = s + bias
            m_cur = jnp.max(s, axis=1, keepdims=True)
            if is_first:
                m_new = m_cur
            else:
                m_prev = m_ref[h][:, 0:1]
                m_new = jnp.maximum(m_prev, m_cur)
            p_ = jnp.exp(s - m_new)
            pv = jax.lax.dot_general(p_, v_ref[h], (((1,), (0,)), ((), ())),
                                     preferred_element_type=jnp.float32)
            if is_first:
                acc_ref[h] = pv
            else:
                alpha = jnp.exp((m_prev - m_new).astype(jnp.float32))
                acc_ref[h] = acc_ref[h] * alpha + pv
            m_ref[h] = jnp.broadcast_to(m_new, (BLK, 128))

    @pl.when(first & full)
    def _ff():
        _run(True, None)

    @pl.when(first & jnp.logical_not(full))
    def _fm():
        _run(True, _make_bias())

    @pl.when(jnp.logical_not(first) & full)
    def _nf():
        _run(False, None)

    @pl.when(jnp.logical_not(first) & jnp.logical_not(full))
    def _nm():
        _run(False, _make_bias())

    @pl.when((p == pl.num_programs(0) - 1)
             | (pi_ref[jnp.minimum(p + 1, NPMAX - 1)] != i))
    def _flush():
        parts = []
        for h in range(N_HEADS):
            a = acc_ref[h]
            inv_l = 1.0 / a[:, DH:DH + 1]
            parts.append((a[:, 0:DH] * inv_l).astype(jnp.bfloat16))
        ctx = jnp.concatenate(parts, axis=1)
        t = jax.lax.dot_general(ctx, wo_ref[...], (((1,), (0,)), ((), ())),
                                preferred_element_type=jnp.float32)
        t = x_ref[...] + t + bo_ref[...]
        x1 = _ln(t, g1_ref[...], b1n_ref[...])
        hh = jax.lax.dot_general(x1.astype(jnp.bfloat16), w1_ref[...],
                                 (((1,), (0,)), ((), ())),
                                 preferred_element_type=jnp.float32)
        hh = jnp.maximum(hh + b1_ref[...], 0.0)
        ff = jax.lax.dot_general(hh.astype(jnp.bfloat16), w2_ref[...],
                                 (((1,), (0,)), ((), ())),
                                 preferred_element_type=jnp.float32)
        ff = ff + b2_ref[...]
        o_ref[...] = _ln(x1 + ff, g2_ref[...], b2n_ref[...])


def _attention(q, k, v, seg3, pair_i, pair_j, pair_f, n_pairs, x, wo, bo,
               g1, b1n, w1, b1, w2, b2, g2, b2n):
    def qmap(p, pi_ref, pj_ref, pf_ref):
        return (0, pi_ref[p], 0)

    def kvmap(p, pi_ref, pj_ref, pf_ref):
        return (0, pj_ref[p], 0)

    def omap(p, pi_ref, pj_ref, pf_ref):
        return (pi_ref[p], 0)

    def sqmap(p, pi_ref, pj_ref, pf_ref):
        return (pi_ref[p], 0, 0)

    def skmap(p, pi_ref, pj_ref, pf_ref):
        return (pj_ref[p], 0, 0)

    def full(p, pi_ref, pj_ref, pf_ref):
        return (0, 0)

    grid_spec = pltpu.PrefetchScalarGridSpec(
        num_scalar_prefetch=3,
        grid=(n_pairs,),
        in_specs=[
            pl.BlockSpec((N_HEADS, BLK, DH), qmap),
            pl.BlockSpec((N_HEADS, BLK, DH), kvmap),
            pl.BlockSpec((N_HEADS, BLK, 128), kvmap),
            pl.BlockSpec((1, 1, BLK), sqmap),
            pl.BlockSpec((1, 1, BLK), skmap),
            pl.BlockSpec((BLK, D_MODEL), omap),
            pl.BlockSpec((D_MODEL, D_MODEL), full),
            pl.BlockSpec((1, D_MODEL), full),
            pl.BlockSpec((1, D_MODEL), full),
            pl.BlockSpec((1, D_MODEL), full),
            pl.BlockSpec((D_MODEL, D_FF), full),
            pl.BlockSpec((1, D_FF), full),
            pl.BlockSpec((D_FF, D_MODEL), full),
            pl.BlockSpec((1, D_MODEL), full),
            pl.BlockSpec((1, D_MODEL), full),
            pl.BlockSpec((1, D_MODEL), full),
        ],
        out_specs=pl.BlockSpec((BLK, D_MODEL), omap),
        scratch_shapes=[
            pltpu.VMEM((N_HEADS, BLK, 128), jnp.float32),
            pltpu.VMEM((N_HEADS, BLK, 128), jnp.bfloat16),
        ],
    )
    return pl.pallas_call(
        _attn_kernel,
        grid_spec=grid_spec,
        out_shape=jax.ShapeDtypeStruct((N_REL, D_MODEL), jnp.float32),
        compiler_params=pltpu.CompilerParams(
            dimension_semantics=("arbitrary",),
        ),
    )(pair_i, pair_j, pair_f, q, k, v, seg3, seg3, x, wo, bo,
      g1, b1n, w1, b1, w2, b2, g2, b2n)


def _ln(x, g, b):
    m = jnp.mean(x, axis=-1, keepdims=True)
    c = x - m
    v = jnp.mean(c * c, axis=-1, keepdims=True)
    return c * jax.lax.rsqrt(v + 1e-5) * g + b


def kernel(rel_features, im_indices, Wq, bq, Wk, bk, Wv, bv, Wo, bo,
           ln1_g, ln1_b, W1, b1, W2, b2, ln2_g, ln2_b):
    seg = im_indices.astype(jnp.int32)
    pair_i, pair_j, pair_f, n_pairs = _compute_pairs(seg)
    seg3 = seg.reshape(NBLK, 1, BLK)

    w_qkv = jnp.concatenate([Wq, Wk, Wv], axis=1).astype(jnp.bfloat16)
    b_qkv = jnp.concatenate([bq, bk, bv]).reshape(1, 3 * D_MODEL)
    q, k, v = _qkv_proj(rel_features, w_qkv, b_qkv)

    out = _attention(q, k, v, seg3, pair_i, pair_j, pair_f, n_pairs,
                     rel_features,
                     Wo.astype(jnp.bfloat16), bo.reshape(1, D_MODEL),
                     ln1_g.reshape(1, D_MODEL), ln1_b.reshape(1, D_MODEL),
                     W1.astype(jnp.bfloat16), b1.reshape(1, D_FF),
                     W2.astype(jnp.bfloat16), b2.reshape(1, D_MODEL),
                     ln2_g.reshape(1, D_MODEL), ln2_b.reshape(1, D_MODEL))
    return out
